# Initial kernel scaffold; baseline (speedup 1.0000x reference)
import jax
import jax.numpy as jnp
from jax.experimental import pallas as pl


def kernel(node_feature, edge_weight, W_mlp, b_mlp, W_ih, b_ih, W_hh, b_hh, edge_index, edge_type):
    raise NotImplementedError("write your pallas kernel here")


# speedup vs baseline: 3.1993x; 3.1993x over previous
import functools

import jax
import jax.numpy as jnp
from jax import lax
from jax.experimental import pallas as pl
from jax.experimental.pallas import tpu as pltpu
from jax.experimental.pallas import tpu_sc as plsc

N = 10000
E = 320000
D = 128
H = 128
T = 3

NC = 2
NS = 16
L = 16
NW = NC * NS

NPAD = 10240
STRIPE = NPAD // NS

CHUNK = 128
EPT = 10112
NCHUNK = EPT // CHUNK
E_PAD = NW * EPT

BLK = 1000
NB = N // BLK


def _tc_pre(x, W_mlp):
    dn = (((1,), (1,)), ((), ()))

    def body(x_ref, wm_ref, y_ref):
        xb = x_ref[...]
        for t in range(T):
            y_ref[t, :, :] = lax.dot_general(
                xb, wm_ref[:, t * D:(t + 1) * D], dn,
                preferred_element_type=jnp.float32)

    return pl.pallas_call(
        body,
        grid=(NB,),
        in_specs=[
            pl.BlockSpec((BLK, D), lambda i: (i, 0)),
            pl.BlockSpec((H, T * D), lambda i: (0, 0)),
        ],
        out_specs=pl.BlockSpec((T, BLK, D), lambda i: (0, i, 0)),
        out_shape=jax.ShapeDtypeStruct((T, N, D), jnp.float32),
    )(x, W_mlp)


def _sc_aggregate(y_table, src, dst, ety, w):
    mesh = plsc.VectorSubcoreMesh(core_axis_name="c", subcore_axis_name="s")

    @functools.partial(
        pl.kernel,
        out_type=jax.ShapeDtypeStruct((NC, NPAD, D), jnp.float32),
        mesh=mesh,
        scratch_types=[
            pltpu.VMEM((CHUNK,), jnp.int32),
            pltpu.VMEM((CHUNK,), jnp.int32),
            pltpu.VMEM((CHUNK,), jnp.int32),
            pltpu.VMEM((CHUNK,), jnp.float32),
            pltpu.VMEM((CHUNK,), jnp.int32),
            pltpu.VMEM((CHUNK, D), jnp.float32),
            pltpu.VMEM_SHARED((NPAD, D), jnp.float32),
        ],
    )
    def agg(y_hbm, src_hbm, dst_hbm, ety_hbm, w_hbm, out_hbm,
            src_v, dst_v, ety_v, w_v, gidx_v, rows_v, acc_sh):
        c = lax.axis_index("c")
        s = lax.axis_index("s")
        wid = c * NS + s

        @pl.loop(0, CHUNK)
        def _(r):
            for j in range(D // L):
                rows_v[r, pl.ds(j * L, L)] = jnp.zeros((L,), jnp.float32)

        for m in range(STRIPE // CHUNK):
            pltpu.sync_copy(rows_v,
                            acc_sh.at[pl.ds(s * STRIPE + m * CHUNK, CHUNK)])
        plsc.subcore_barrier()

        @pl.loop(0, NCHUNK)
        def _(k):
            e0 = wid * EPT + k * CHUNK
            pltpu.sync_copy(src_hbm.at[pl.ds(e0, CHUNK)], src_v)
            pltpu.sync_copy(dst_hbm.at[pl.ds(e0, CHUNK)], dst_v)
            pltpu.sync_copy(ety_hbm.at[pl.ds(e0, CHUNK)], ety_v)
            pltpu.sync_copy(w_hbm.at[pl.ds(e0, CHUNK)], w_v)
            for v in range(CHUNK // L):
                gidx_v[pl.ds(v * L, L)] = (ety_v[pl.ds(v * L, L)] * N
                                           + src_v[pl.ds(v * L, L)])
            pltpu.sync_copy(y_hbm.at[gidx_v], rows_v)

            @pl.loop(0, CHUNK // L)
            def _(g):
                wvec = w_v[pl.ds(g * L, L)]
                for i in range(L):
                    wi = wvec[i]
                    r0 = g * L + i
                    for j in range(D // L):
                        rows_v[r0, pl.ds(j * L, L)] = (
                            rows_v[r0, pl.ds(j * L, L)] * wi)

            pltpu.sync_copy(rows_v, acc_sh.at[dst_v], add=True)

        plsc.subcore_barrier()
        pltpu.sync_copy(acc_sh.at[pl.ds(s * STRIPE, STRIPE)],
                        out_hbm.at[c, pl.ds(s * STRIPE, STRIPE)])

    return agg(y_table, src, dst, ety, w)


def _tc_post(acc, x, b_mlp, W_ih, b_ih, W_hh, b_hh):
    dn = (((1,), (1,)), ((), ()))

    def body(a0_ref, a1_ref, x_ref, bm_ref,
             wih_ref, bih_ref, whh_ref, bhh_ref, o_ref):
        xb = x_ref[...]
        hidden = jnp.maximum(a0_ref[0] + a1_ref[0] + bm_ref[...], 0.0)

        def gates(t):
            gi = lax.dot_general(hidden, wih_ref[t * D:(t + 1) * D, :], dn,
                                 preferred_element_type=jnp.float32)
            gh = lax.dot_general(xb, whh_ref[t * D:(t + 1) * D, :], dn,
                                 preferred_element_type=jnp.float32)
            return gi + bih_ref[t:t + 1, :], gh + bhh_ref[t:t + 1, :]

        ir, hr = gates(0)
        iz, hz = gates(1)
        in_, hn = gates(2)
        r = jax.nn.sigmoid(ir + hr)
        z = jax.nn.sigmoid(iz + hz)
        n = jnp.tanh(in_ + r * hn)
        o_ref[...] = (1.0 - z) * n + z * xb

    def aspec(cidx):
        return pl.BlockSpec((1, BLK, D), lambda i, c=cidx: (c, i, 0))

    return pl.pallas_call(
        body,
        grid=(NB,),
        in_specs=[
            aspec(0), aspec(1),
            pl.BlockSpec((BLK, D), lambda i: (i, 0)),
            pl.BlockSpec((1, H), lambda i: (0, 0)),
            pl.BlockSpec((T * D, H), lambda i: (0, 0)),
            pl.BlockSpec((T, D), lambda i: (0, 0)),
            pl.BlockSpec((T * D, D), lambda i: (0, 0)),
            pl.BlockSpec((T, D), lambda i: (0, 0)),
        ],
        out_specs=pl.BlockSpec((BLK, D), lambda i: (i, 0)),
        out_shape=jax.ShapeDtypeStruct((N, D), jnp.float32),
    )(acc, acc, x, b_mlp.reshape(1, H), W_ih, b_ih.reshape(T, D),
      W_hh, b_hh.reshape(T, D))


def kernel(node_feature, edge_weight, W_mlp, b_mlp, W_ih, b_ih, W_hh, b_hh,
           edge_index, edge_type):
    src = edge_index[0].astype(jnp.int32)
    dst = edge_index[1].astype(jnp.int32)
    ety = edge_type.astype(jnp.int32)
    w = edge_weight.astype(jnp.float32)
    pad = E_PAD - E
    src = jnp.concatenate([src, jnp.zeros((pad,), jnp.int32)])
    dst = jnp.concatenate([dst, jnp.zeros((pad,), jnp.int32)])
    ety = jnp.concatenate([ety, jnp.zeros((pad,), jnp.int32)])
    w = jnp.concatenate([w, jnp.zeros((pad,), jnp.float32)])

    y = _tc_pre(node_feature, W_mlp).reshape(T * N, D)
    acc = _sc_aggregate(y, src, dst, ety, w)
    return _tc_post(acc, node_feature, b_mlp, W_ih, b_ih, W_hh, b_hh)


# speedup vs baseline: 3.4371x; 1.0743x over previous
import functools

import jax
import jax.numpy as jnp
from jax import lax
from jax.experimental import pallas as pl
from jax.experimental.pallas import tpu as pltpu
from jax.experimental.pallas import tpu_sc as plsc

N = 10000
E = 320000
D = 128
H = 128
T = 3

NC = 2
NS = 16
L = 16
NW = NC * NS

NPAD = 10240
STRIPE = NPAD // NS

CHUNK = 128
CPT = 80
EPT = CPT * CHUNK
E_PAD = NW * EPT
NROW = E_PAD // CHUNK
NROW_PAD = NROW + 8
NSUPER = CPT // 8

ZR = 32

BLK = 1000
NB = N // BLK


def _tc_pre(x, W_mlp):
    dn = (((1,), (1,)), ((), ()))

    def body(x_ref, wm_ref, y_ref):
        xb = x_ref[...]
        for t in range(T):
            y_ref[t, :, :] = lax.dot_general(
                xb, wm_ref[:, t * D:(t + 1) * D], dn,
                preferred_element_type=jnp.float32)

    return pl.pallas_call(
        body,
        grid=(NB,),
        in_specs=[
            pl.BlockSpec((BLK, D), lambda i: (i, 0)),
            pl.BlockSpec((H, T * D), lambda i: (0, 0)),
        ],
        out_specs=pl.BlockSpec((T, BLK, D), lambda i: (0, i, 0)),
        out_shape=jax.ShapeDtypeStruct((T, N, D), jnp.float32),
    )(x, W_mlp)


def _sc_aggregate(y_table, packed, warr):
    mesh = plsc.VectorSubcoreMesh(core_axis_name="c", subcore_axis_name="s")

    @functools.partial(
        pl.kernel,
        out_type=jax.ShapeDtypeStruct((NC, NPAD, D), jnp.float32),
        mesh=mesh,
        scratch_types=[
            pltpu.VMEM((8, CHUNK), jnp.int32),
            pltpu.VMEM((8, CHUNK), jnp.float32),
            *[pltpu.VMEM((CHUNK,), jnp.int32) for _ in range(8)],
            *[pltpu.VMEM((CHUNK,), jnp.int32) for _ in range(8)],
            pltpu.VMEM((ZR, D), jnp.float32),
            pltpu.VMEM((CHUNK, D), jnp.float32),
            pltpu.VMEM((CHUNK, D), jnp.float32),
            pltpu.SemaphoreType.DMA,
            pltpu.SemaphoreType.DMA,
            pltpu.SemaphoreType.DMA,
            pltpu.SemaphoreType.DMA,
            pltpu.SemaphoreType.DMA,
            pltpu.VMEM_SHARED((NPAD, D), jnp.float32),
        ],
    )
    def agg(y_hbm, pk_hbm, w_hbm, out_hbm,
            pk8, w8, g0, g1, g2, g3, g4, g5, g6, g7,
            d0, d1, d2, d3, d4, d5, d6, d7, zbuf, rows0, rows1,
            sem_i, sem_g0, sem_g1, sem_s0, sem_s1, acc_sh):
        c = lax.axis_index("c")
        s = lax.axis_index("s")
        wid = c * NS + s
        tbase = wid * CPT

        gidx = (g0, g1, g2, g3, g4, g5, g6, g7)
        dst = (d0, d1, d2, d3, d4, d5, d6, d7)
        rows = (rows0, rows1)
        sem_g = (sem_g0, sem_g1)
        sem_s = (sem_s0, sem_s1)

        @pl.loop(0, ZR)
        def _(r):
            for j in range(D // L):
                zbuf[r, pl.ds(j * L, L)] = jnp.zeros((L,), jnp.float32)

        for m in range(STRIPE // ZR):
            pltpu.sync_copy(zbuf, acc_sh.at[pl.ds(s * STRIPE + m * ZR, ZR)])
        plsc.subcore_barrier()

        def decode(u):
            for v in range(CHUNK // L):
                pk = pk8[u, pl.ds(v * L, L)]
                gidx[u][pl.ds(v * L, L)] = jnp.minimum(
                    pk & jnp.int32(0xFFFF), jnp.int32(T * N - 1))
                dst[u][pl.ds(v * L, L)] = lax.shift_right_logical(pk, 16)

        def scale(b, row):
            @pl.loop(0, CHUNK // L)
            def _(g):
                wvec = w8[row, pl.ds(g * L, L)]
                for i in range(L):
                    wi = wvec[i]
                    r0 = g * L + i
                    for j in range(D // L):
                        rows[b][r0, pl.ds(j * L, L)] = (
                            rows[b][r0, pl.ds(j * L, L)] * wi)

        @pl.loop(0, NSUPER)
        def _(q):
            c0 = tbase + q * 8
            hp = pltpu.async_copy(pk_hbm.at[pl.ds(c0, 8)], pk8, sem_i)
            hw = pltpu.async_copy(w_hbm.at[pl.ds(c0, 8)], w8, sem_i)
            hp.wait()
            hw.wait()
            decode(0)
            hg = [None] * 8
            hs = [None] * 8
            hg[0] = pltpu.async_copy(y_hbm.at[gidx[0]], rows0, sem_g0)
            for u in range(1, 8):
                decode(u)
            for u in range(8):
                b = u % 2
                b2 = (u + 1) % 2
                hg[u].wait()
                if u >= 1:
                    hs[u - 1].wait()
                if u + 1 < 8:
                    hg[u + 1] = pltpu.async_copy(y_hbm.at[gidx[u + 1]],
                                                 rows[b2], sem_g[b2])
                scale(b, u)
                hs[u] = pltpu.async_copy(rows[b], acc_sh.at[dst[u]],
                                         sem_s[b], add=True)
            hs[7].wait()

        plsc.subcore_barrier()
        pltpu.sync_copy(acc_sh.at[pl.ds(s * STRIPE, STRIPE)],
                        out_hbm.at[c, pl.ds(s * STRIPE, STRIPE)])

    return agg(y_table, packed, warr)


def _tc_post(acc, x, b_mlp, W_ih, b_ih, W_hh, b_hh):
    dn = (((1,), (1,)), ((), ()))

    def body(a0_ref, a1_ref, x_ref, bm_ref,
             wih_ref, bih_ref, whh_ref, bhh_ref, o_ref):
        xb = x_ref[...]
        hidden = jnp.maximum(a0_ref[0] + a1_ref[0] + bm_ref[...], 0.0)

        def gates(t):
            gi = lax.dot_general(hidden, wih_ref[t * D:(t + 1) * D, :], dn,
                                 preferred_element_type=jnp.float32)
            gh = lax.dot_general(xb, whh_ref[t * D:(t + 1) * D, :], dn,
                                 preferred_element_type=jnp.float32)
            return gi + bih_ref[t:t + 1, :], gh + bhh_ref[t:t + 1, :]

        ir, hr = gates(0)
        iz, hz = gates(1)
        in_, hn = gates(2)
        r = jax.nn.sigmoid(ir + hr)
        z = jax.nn.sigmoid(iz + hz)
        n = jnp.tanh(in_ + r * hn)
        o_ref[...] = (1.0 - z) * n + z * xb

    def aspec(cidx):
        return pl.BlockSpec((1, BLK, D), lambda i, c=cidx: (c, i, 0))

    return pl.pallas_call(
        body,
        grid=(NB,),
        in_specs=[
            aspec(0), aspec(1),
            pl.BlockSpec((BLK, D), lambda i: (i, 0)),
            pl.BlockSpec((1, H), lambda i: (0, 0)),
            pl.BlockSpec((T * D, H), lambda i: (0, 0)),
            pl.BlockSpec((T, D), lambda i: (0, 0)),
            pl.BlockSpec((T * D, D), lambda i: (0, 0)),
            pl.BlockSpec((T, D), lambda i: (0, 0)),
        ],
        out_specs=pl.BlockSpec((BLK, D), lambda i: (i, 0)),
        out_shape=jax.ShapeDtypeStruct((N, D), jnp.float32),
    )(acc, acc, x, b_mlp.reshape(1, H), W_ih, b_ih.reshape(T, D),
      W_hh, b_hh.reshape(T, D))


def kernel(node_feature, edge_weight, W_mlp, b_mlp, W_ih, b_ih, W_hh, b_hh,
           edge_index, edge_type):
    src = edge_index[0].astype(jnp.int32)
    dst = edge_index[1].astype(jnp.int32)
    ety = edge_type.astype(jnp.int32)
    w = edge_weight.astype(jnp.float32)
    pk = jnp.left_shift(dst, 16) | (ety * N + src)
    npad = NROW_PAD * CHUNK - E
    pk = jnp.concatenate([pk, jnp.zeros((npad,), jnp.int32)])
    w = jnp.concatenate([w, jnp.zeros((npad,), jnp.float32)])
    pk = pk.reshape(NROW_PAD, CHUNK)
    w = w.reshape(NROW_PAD, CHUNK)

    y = _tc_pre(node_feature, W_mlp).reshape(T * N, D)
    acc = _sc_aggregate(y, pk, w)
    return _tc_post(acc, node_feature, b_mlp, W_ih, b_ih, W_hh, b_hh)


# speedup vs baseline: 9.0738x; 2.6400x over previous
import functools

import jax
import jax.numpy as jnp
from jax import lax
from jax.experimental import pallas as pl
from jax.experimental.pallas import tpu as pltpu
from jax.experimental.pallas import tpu_sc as plsc

N = 10000
E = 320000
D = 128
H = 128
T = 3

NC = 2
NS = 16
L = 16
NW = NC * NS

NPAD = 10240
STRIPE = NPAD // NS

CHUNK = 128
CPT = 80
EPT = CPT * CHUNK
E_PAD = NW * EPT
NROW = E_PAD // CHUNK
NROW_PAD = NROW + 8
NSUPER = CPT // 8

ZR = 32

BLK = 1000
NB = N // BLK


def _tc_pre(x, W_mlp):
    dn = (((1,), (1,)), ((), ()))

    def body(x_ref, wm_ref, y_ref):
        xb = x_ref[...]
        for t in range(T):
            y_ref[t, :, :] = lax.dot_general(
                xb, wm_ref[:, t * D:(t + 1) * D], dn,
                preferred_element_type=jnp.float32)

    return pl.pallas_call(
        body,
        grid=(NB,),
        in_specs=[
            pl.BlockSpec((BLK, D), lambda i: (i, 0)),
            pl.BlockSpec((H, T * D), lambda i: (0, 0)),
        ],
        out_specs=pl.BlockSpec((T, BLK, D), lambda i: (0, i, 0)),
        out_shape=jax.ShapeDtypeStruct((T, N, D), jnp.float32),
    )(x, W_mlp)


def _sc_aggregate(y_table, packed, warr):
    mesh = plsc.VectorSubcoreMesh(core_axis_name="c", subcore_axis_name="s")

    @functools.partial(
        pl.kernel,
        out_type=jax.ShapeDtypeStruct((NC, NPAD, D), jnp.float32),
        mesh=mesh,
        scratch_types=[
            pltpu.VMEM((8, CHUNK), jnp.int32),
            pltpu.VMEM((8, CHUNK), jnp.float32),
            *[pltpu.VMEM((CHUNK,), jnp.int32) for _ in range(8)],
            *[pltpu.VMEM((CHUNK,), jnp.int32) for _ in range(8)],
            pltpu.VMEM((ZR, D), jnp.float32),
            pltpu.VMEM((CHUNK, D), jnp.float32),
            pltpu.VMEM((CHUNK, D), jnp.float32),
            pltpu.SemaphoreType.DMA,
            pltpu.SemaphoreType.DMA,
            pltpu.SemaphoreType.DMA,
            pltpu.SemaphoreType.DMA,
            pltpu.SemaphoreType.DMA,
            pltpu.VMEM_SHARED((NPAD, D), jnp.float32),
        ],
    )
    def agg(y_hbm, pk_hbm, w_hbm, out_hbm,
            pk8, w8, g0, g1, g2, g3, g4, g5, g6, g7,
            d0, d1, d2, d3, d4, d5, d6, d7, zbuf, rows0, rows1,
            sem_i, sem_g0, sem_g1, sem_s0, sem_s1, acc_sh):
        c = lax.axis_index("c")
        s = lax.axis_index("s")
        wid = c * NS + s
        tbase = wid * CPT

        gidx = (g0, g1, g2, g3, g4, g5, g6, g7)
        dst = (d0, d1, d2, d3, d4, d5, d6, d7)
        rows = (rows0, rows1)
        sem_g = (sem_g0, sem_g1)
        sem_s = (sem_s0, sem_s1)

        @pl.loop(0, ZR)
        def _(r):
            for j in range(D // L):
                zbuf[r, pl.ds(j * L, L)] = jnp.zeros((L,), jnp.float32)

        for m in range(STRIPE // ZR):
            pltpu.sync_copy(zbuf, acc_sh.at[pl.ds(s * STRIPE + m * ZR, ZR)])
        plsc.subcore_barrier()

        def decode(u):
            for v in range(CHUNK // L):
                pk = pk8[u, pl.ds(v * L, L)]
                gidx[u][pl.ds(v * L, L)] = jnp.minimum(
                    pk & jnp.int32(0xFFFF), jnp.int32(T * N - 1))
                dst[u][pl.ds(v * L, L)] = lax.shift_right_logical(pk, 16)

        def scale(b, row):
            @pl.loop(0, CHUNK // L)
            def _(g):
                wvec = w8[row, pl.ds(g * L, L)]
                for i in range(L):
                    wi = wvec[i]
                    r0 = g * L + i
                    for j in range(D // L):
                        rows[b][r0, pl.ds(j * L, L)] = (
                            rows[b][r0, pl.ds(j * L, L)] * wi)

        @pl.loop(0, NSUPER)
        def _(q):
            c0 = tbase + q * 8
            hp = pltpu.async_copy(pk_hbm.at[pl.ds(c0, 8)], pk8, sem_i)
            hw = pltpu.async_copy(w_hbm.at[pl.ds(c0, 8)], w8, sem_i)
            hp.wait()
            hw.wait()
            decode(0)
            hg = [None] * 8
            hs = [None] * 8
            hg[0] = pltpu.async_copy(y_hbm.at[gidx[0]], rows0, sem_g0)
            for u in range(1, 8):
                decode(u)
            for u in range(8):
                b = u % 2
                b2 = (u + 1) % 2
                hg[u].wait()
                if u >= 1:
                    hs[u - 1].wait()
                if u + 1 < 8:
                    hg[u + 1] = pltpu.async_copy(y_hbm.at[gidx[u + 1]],
                                                 rows[b2], sem_g[b2])
                scale(b, u)
                hs[u] = pltpu.async_copy(rows[b], acc_sh.at[dst[u]],
                                         sem_s[b], add=True)
            hs[7].wait()

        plsc.subcore_barrier()
        pltpu.sync_copy(acc_sh.at[pl.ds(s * STRIPE, STRIPE)],
                        out_hbm.at[c, pl.ds(s * STRIPE, STRIPE)])

    return agg(y_table, packed, warr)


def _tc_post(acc, x, b_mlp, W_ih, b_ih, W_hh, b_hh):
    dn = (((1,), (1,)), ((), ()))

    def body(a0_ref, a1_ref, x_ref, bm_ref,
             wih_ref, bih_ref, whh_ref, bhh_ref, o_ref):
        xb = x_ref[...]
        hidden = jnp.maximum(a0_ref[0] + a1_ref[0] + bm_ref[...], 0.0)

        def gates(t):
            gi = lax.dot_general(hidden, wih_ref[t * D:(t + 1) * D, :], dn,
                                 preferred_element_type=jnp.float32)
            gh = lax.dot_general(xb, whh_ref[t * D:(t + 1) * D, :], dn,
                                 preferred_element_type=jnp.float32)
            return gi + bih_ref[t:t + 1, :], gh + bhh_ref[t:t + 1, :]

        ir, hr = gates(0)
        iz, hz = gates(1)
        in_, hn = gates(2)
        r = jax.nn.sigmoid(ir + hr)
        z = jax.nn.sigmoid(iz + hz)
        n = jnp.tanh(in_ + r * hn)
        o_ref[...] = (1.0 - z) * n + z * xb

    def aspec(cidx):
        return pl.BlockSpec((1, BLK, D), lambda i, c=cidx: (c, i, 0))

    return pl.pallas_call(
        body,
        grid=(NB,),
        in_specs=[
            aspec(0), aspec(1),
            pl.BlockSpec((BLK, D), lambda i: (i, 0)),
            pl.BlockSpec((1, H), lambda i: (0, 0)),
            pl.BlockSpec((T * D, H), lambda i: (0, 0)),
            pl.BlockSpec((T, D), lambda i: (0, 0)),
            pl.BlockSpec((T * D, D), lambda i: (0, 0)),
            pl.BlockSpec((T, D), lambda i: (0, 0)),
        ],
        out_specs=pl.BlockSpec((BLK, D), lambda i: (i, 0)),
        out_shape=jax.ShapeDtypeStruct((N, D), jnp.float32),
    )(acc, acc, x, b_mlp.reshape(1, H), W_ih, b_ih.reshape(T, D),
      W_hh, b_hh.reshape(T, D))


def kernel(node_feature, edge_weight, W_mlp, b_mlp, W_ih, b_ih, W_hh, b_hh,
           edge_index, edge_type):
    src = edge_index[0].astype(jnp.int32)
    dst = edge_index[1].astype(jnp.int32)
    ety = edge_type.astype(jnp.int32)
    w = edge_weight.astype(jnp.float32)
    pk = jnp.left_shift(dst, 16) | (ety * N + src)
    npad = NROW_PAD * CHUNK - E
    spread = jnp.arange(npad, dtype=jnp.int32) % N
    pk = jnp.concatenate([pk, jnp.left_shift(spread, 16) | spread])
    w = jnp.concatenate([w, jnp.zeros((npad,), jnp.float32)])
    pk = pk.reshape(NROW_PAD, CHUNK)
    w = w.reshape(NROW_PAD, CHUNK)

    y = _tc_pre(node_feature, W_mlp).reshape(T * N, D)
    acc = _sc_aggregate(y, pk, w)
    return _tc_post(acc, node_feature, b_mlp, W_ih, b_ih, W_hh, b_hh)
